# baseline (speedup 1.0000x reference)
import functools

import jax
import jax.numpy as jnp
from jax import lax
from jax.experimental import pallas as pl
from jax.experimental.pallas import tpu as pltpu
from jax.experimental.pallas import tpu_sc as plsc

_info = plsc.get_sparse_core_info()
_NC = _info.num_cores
_NS = _info.num_subcores
_NW = _NC * _NS


def _transpose_body(x_ref, o_ref):
  o_ref[...] = x_ref[...].T


@functools.lru_cache(maxsize=None)
def _make_table_rowmajor(vocab, dim, cols):
  grid = (vocab + cols - 1) // cols
  return pl.pallas_call(
      _transpose_body,
      grid=(grid,),
      in_specs=[pl.BlockSpec((dim, cols), lambda i: (0, i))],
      out_specs=pl.BlockSpec((cols, dim), lambda i: (i, 0)),
      out_shape=jax.ShapeDtypeStruct((vocab, dim), jnp.float32),
  )


@functools.lru_cache(maxsize=None)
def _make_gather(vocab, dim, n, chunk, nbuf):
  n_per_w = n // _NW
  n_chunks = n_per_w // chunk
  n_groups = n_chunks // nbuf
  mesh = plsc.VectorSubcoreMesh(core_axis_name="c", subcore_axis_name="s")

  @functools.partial(
      pl.kernel,
      mesh=mesh,
      out_type=jax.ShapeDtypeStruct((n, dim), jnp.float32),
      compiler_params=pltpu.CompilerParams(use_tc_tiling_on_sc=False),
      scratch_types=[
          pltpu.VMEM((n_per_w,), jnp.int32),
          pltpu.VMEM((nbuf, chunk, dim), jnp.float32),
      ]
      + [pltpu.SemaphoreType.DMA] * (2 * nbuf),
  )
  def gather_kernel(table_hbm, idx_hbm, out_hbm, idx_v, rows_v, *sems):
    gsem = sems[:nbuf]
    osem = sems[nbuf:]
    wid = lax.axis_index("s") * _NC + lax.axis_index("c")
    base = wid * n_per_w
    pltpu.sync_copy(idx_hbm.at[pl.ds(base, n_per_w)], idx_v)

    def start_gather(b, c):
      pltpu.async_copy(
          table_hbm.at[idx_v.at[pl.ds(c * chunk, chunk)]],
          rows_v.at[b],
          gsem[b],
      )

    def wait_gather(b):
      pltpu.make_async_copy(
          table_hbm.at[pl.ds(0, chunk)], rows_v.at[b], gsem[b]
      ).wait()

    def start_out(b, c):
      pltpu.async_copy(
          rows_v.at[b], out_hbm.at[pl.ds(base + c * chunk, chunk)], osem[b]
      )

    def wait_out(b):
      pltpu.make_async_copy(
          out_hbm.at[pl.ds(base, chunk)], rows_v.at[b], osem[b]
      ).wait()

    for b in range(nbuf):
      start_gather(b, b)

    def group_body(g, carry):
      c0 = g * nbuf
      for b in range(nbuf):
        wait_gather(b)
        start_out(b, c0 + b)
      for b in range(nbuf):
        wait_out(b)
        start_gather(b, c0 + nbuf + b)
      return carry

    lax.fori_loop(0, n_groups - 1, group_body, 0)

    c0 = (n_groups - 1) * nbuf
    for b in range(nbuf):
      wait_gather(b)
      start_out(b, c0 + b)
    for b in range(nbuf):
      wait_out(b)

  return gather_kernel


def _pack_body(s_blk, dim, x_ref, o_ref):
  for r in range(s_blk):
    y = x_ref[0, :, r, :].T
    o_ref[r] = y.reshape(dim // 8, 1, 8, 128)


@functools.lru_cache(maxsize=None)
def _make_pack(n_bt, seq, dim, s_blk):
  body = functools.partial(_pack_body, s_blk, dim)
  return pl.pallas_call(
      body,
      grid=(n_bt, seq // s_blk),
      in_specs=[
          pl.BlockSpec((1, 128, s_blk, dim), lambda i, j: (i, 0, j, 0))
      ],
      out_specs=pl.BlockSpec(
          (s_blk, dim // 8, 1, 8, 128), lambda i, j: (j, 0, i, 0, 0)
      ),
      out_shape=jax.ShapeDtypeStruct(
          (seq, dim // 8, n_bt, 8, 128), jnp.float32
      ),
  )


def kernel(inp, table):
  b, s = inp.shape
  vocab, dim = table.shape
  n = b * s
  flat = inp.reshape(n).astype(jnp.int32)

  table_rm = _make_table_rowmajor(vocab, dim, 4096)(table.T)

  chunk, nbuf = 640, 5
  pad = (-n) % (_NW * chunk * nbuf)
  if pad:
    flat = jnp.concatenate([flat, jnp.zeros((pad,), jnp.int32)])
  rows = _make_gather(vocab, dim, n + pad, chunk, nbuf)(table_rm, flat)
  if pad:
    rows = rows[:n]

  if b % 128 == 0 and s % 8 == 0 and dim % 8 == 0:
    rows4 = rows.reshape(b // 128, 128, s, dim)
    out5 = _make_pack(b // 128, s, dim, 8)(rows4)
    out = out5.transpose(2, 4, 0, 1, 3).reshape(b, s, dim)
  else:
    out = rows.reshape(b, s, dim)
  return out


# speedup vs baseline: 1.5984x; 1.5984x over previous
import functools

import jax
import jax.numpy as jnp
from jax import lax
from jax.experimental import pallas as pl
from jax.experimental.pallas import tpu as pltpu
from jax.experimental.pallas import tpu_sc as plsc

_info = plsc.get_sparse_core_info()
_NC = _info.num_cores
_NS = _info.num_subcores
_NW = _NC * _NS


@functools.lru_cache(maxsize=None)
def _make_gather_pack(vocab, seq, n_bt, dim):
  st_n = seq // 8
  n_h = dim // 16
  mesh = plsc.VectorSubcoreMesh(core_axis_name="c", subcore_axis_name="s")

  @functools.partial(
      pl.kernel,
      mesh=mesh,
      out_type=jax.ShapeDtypeStruct((seq, dim // 8, n_bt, 8, 128),
                                    jnp.float32),
      compiler_params=pltpu.CompilerParams(
          use_tc_tiling_on_sc=False, needs_layout_passes=False
      ),
      scratch_types=[
          pltpu.VMEM((st_n, 8, 128), jnp.int32),
          pltpu.VMEM((2, 128, dim), jnp.float32),
          pltpu.VMEM((2, dim // 8, 8, 128), jnp.float32),
      ]
      + [pltpu.SemaphoreType.DMA] * 4,
  )
  def gather_pack(table_hbm, inp4_hbm, out_hbm, idx_v, rows_v, t_v, *sems):
    gsem = sems[:2]
    osem = sems[2:]
    w = lax.axis_index("s") * _NC + lax.axis_index("c")
    pltpu.sync_copy(inp4_hbm.at[pl.ds(0, st_n), w], idx_v)

    lane = jax.lax.iota(jnp.int32, 16)
    dr = lane & 7
    g_h = [(lane >> 3) + 2 * h for h in range(n_h)]
    bc_t = [jnp.full((16,), t, jnp.int32) for t in range(128)]

    def start_gather(p, s):
      st = s // 8
      r = s % 8
      pltpu.async_copy(
          table_hbm.at[idx_v.at[st, r]], rows_v.at[p], gsem[p]
      )

    def wait_gather(p):
      pltpu.make_async_copy(
          table_hbm.at[pl.ds(0, 128)], rows_v.at[p], gsem[p]
      ).wait()

    def start_write(p, s):
      pltpu.async_copy(t_v.at[p], out_hbm.at[s, pl.ds(0, dim // 8), w],
                       osem[p])

    def wait_write(p):
      pltpu.make_async_copy(
          out_hbm.at[0, pl.ds(0, dim // 8), 0], t_v.at[p], osem[p]
      ).wait()

    start_gather(0, 0)
    start_gather(1, 1)

    def body(s2, carry):
      for p in range(2):
        s = 2 * s2 + p
        wait_gather(p)

        @pl.when(s2 >= 1)
        def _():
          wait_write(p)

        for t in range(128):
          for h in range(n_h):
            x = rows_v[p, t, pl.ds(16 * h, 16)]
            plsc.store_scatter(t_v.at[p], [g_h[h], dr, bc_t[t]], x)

        @pl.when(s2 < seq // 2 - 1)
        def _():
          start_gather(p, s + 2)

        start_write(p, s)
      return carry

    lax.fori_loop(0, seq // 2, body, 0)
    wait_write(0)
    wait_write(1)

  return gather_pack


@functools.lru_cache(maxsize=None)
def _make_gather(vocab, dim, n, chunk, nbuf):
  n_per_w = n // _NW
  n_chunks = n_per_w // chunk
  n_groups = n_chunks // nbuf
  mesh = plsc.VectorSubcoreMesh(core_axis_name="c", subcore_axis_name="s")

  @functools.partial(
      pl.kernel,
      mesh=mesh,
      out_type=jax.ShapeDtypeStruct((n, dim), jnp.float32),
      compiler_params=pltpu.CompilerParams(use_tc_tiling_on_sc=False),
      scratch_types=[
          pltpu.VMEM((n_per_w,), jnp.int32),
          pltpu.VMEM((nbuf, chunk, dim), jnp.float32),
      ]
      + [pltpu.SemaphoreType.DMA] * (2 * nbuf),
  )
  def gather_kernel(table_hbm, idx_hbm, out_hbm, idx_v, rows_v, *sems):
    gsem = sems[:nbuf]
    osem = sems[nbuf:]
    wid = lax.axis_index("s") * _NC + lax.axis_index("c")
    base = wid * n_per_w
    pltpu.sync_copy(idx_hbm.at[pl.ds(base, n_per_w)], idx_v)

    def start_gather(b, c):
      pltpu.async_copy(
          table_hbm.at[idx_v.at[pl.ds(c * chunk, chunk)]],
          rows_v.at[b],
          gsem[b],
      )

    def wait_gather(b):
      pltpu.make_async_copy(
          table_hbm.at[pl.ds(0, chunk)], rows_v.at[b], gsem[b]
      ).wait()

    def start_out(b, c):
      pltpu.async_copy(
          rows_v.at[b], out_hbm.at[pl.ds(base + c * chunk, chunk)], osem[b]
      )

    def wait_out(b):
      pltpu.make_async_copy(
          out_hbm.at[pl.ds(base, chunk)], rows_v.at[b], osem[b]
      ).wait()

    for b in range(nbuf):
      start_gather(b, b)

    def group_body(g, carry):
      c0 = g * nbuf
      for b in range(nbuf):
        wait_gather(b)
        start_out(b, c0 + b)
      for b in range(nbuf):
        wait_out(b)
        start_gather(b, c0 + nbuf + b)
      return carry

    lax.fori_loop(0, n_groups - 1, group_body, 0)

    c0 = (n_groups - 1) * nbuf
    for b in range(nbuf):
      wait_gather(b)
      start_out(b, c0 + b)
    for b in range(nbuf):
      wait_out(b)

  return gather_kernel


def kernel(inp, table):
  b, s = inp.shape
  vocab, dim = table.shape
  n = b * s
  inp = inp.astype(jnp.int32)

  if b == 128 * _NW and s % 8 == 0 and dim % 16 == 0 and dim >= 16:
    n_bt = b // 128
    inp4 = (
        inp.T.reshape(s // 8, 8, n_bt, 128).transpose(0, 2, 1, 3)
    )
    out5 = _make_gather_pack(vocab, s, n_bt, dim)(table, inp4)
    return out5.transpose(2, 4, 0, 1, 3).reshape(b, s, dim)

  flat = inp.reshape(n)
  chunk, nbuf = 640, 5
  pad = (-n) % (_NW * chunk * nbuf)
  if pad:
    flat = jnp.concatenate([flat, jnp.zeros((pad,), jnp.int32)])
  rows = _make_gather(vocab, dim, n + pad, chunk, nbuf)(table, flat)
  if pad:
    rows = rows[:n]
  return rows.reshape(b, s, dim)


# speedup vs baseline: 3.2591x; 2.0389x over previous
import functools

import jax
import jax.numpy as jnp
from jax import lax
from jax.experimental import pallas as pl
from jax.experimental.pallas import tpu as pltpu
from jax.experimental.pallas import tpu_sc as plsc

_info = plsc.get_sparse_core_info()
_NC = _info.num_cores
_NS = _info.num_subcores
_NW = _NC * _NS


def _iota16():
  return jax.lax.iota(jnp.int32, 16)


@functools.lru_cache(maxsize=None)
def _make_table_linear(vocab, dim):
  nblk = vocab // 128
  rem = vocab - nblk * 128
  per_w = nblk // _NW
  extra = nblk - per_w * _NW
  mesh = plsc.VectorSubcoreMesh(core_axis_name="c", subcore_axis_name="s")

  @functools.partial(
      pl.kernel,
      mesh=mesh,
      out_type=jax.ShapeDtypeStruct((vocab * dim,), jnp.float32),
      compiler_params=pltpu.CompilerParams(
          use_tc_tiling_on_sc=True, needs_layout_passes=False
      ),
      scratch_types=[
          pltpu.VMEM((dim, 128), jnp.float32),
          pltpu.VMEM((dim, 128), jnp.float32),
          pltpu.VMEM((128 * dim,), jnp.float32),
          pltpu.VMEM((128 * dim,), jnp.float32),
      ]
      + [pltpu.SemaphoreType.DMA] * 4,
  )
  def detile(tt_hbm, tail_hbm, out_hbm, tile_v0, tile_v1, lin_v0, lin_v1,
             *sems):
    tile_v = (tile_v0, tile_v1)
    lin_v = (lin_v0, lin_v1)
    isem = sems[:2]
    osem = sems[2:]
    w = lax.axis_index("s") * _NC + lax.axis_index("c")
    iota = _iota16()
    perm = [(iota + k) & 15 for k in range(16)]
    vk = [(((iota + k) & 15) * dim) + iota for k in range(16)]

    def c0_of(j):
      return (j * _NW + w) * 128

    def start_in(p, j):
      pltpu.async_copy(
          tt_hbm.at[pl.ds(0, dim), pl.ds(c0_of(j), 128)],
          tile_v[p],
          isem[p],
      )

    def wait_in(p):
      pltpu.make_async_copy(
          tt_hbm.at[pl.ds(0, dim), pl.ds(0, 128)], tile_v[p], isem[p]
      ).wait()

    def start_out(p, j):
      pltpu.async_copy(
          lin_v[p], out_hbm.at[pl.ds(c0_of(j) * dim, 128 * dim)], osem[p]
      )

    def wait_out(p):
      pltpu.make_async_copy(
          out_hbm.at[pl.ds(0, 128 * dim)], lin_v[p], osem[p]
      ).wait()

    def transpose_block(p, width):
      def strip(ci, carry):
        cl0 = ci * 16
        for d0 in range(0, dim, 16):
          base = cl0 * dim + d0
          for k in range(16):
            x = plsc.load_gather(
                tile_v[p], [iota + d0, perm[k] + cl0]
            )
            plsc.store_scatter(lin_v[p], [vk[k] + base], x)
        return carry

      lax.fori_loop(0, width // 16, strip, 0)

    start_in(0, 0)
    start_in(1, 1)

    def body(j2, carry):
      for p in range(2):
        j = 2 * j2 + p
        wait_in(p)

        @pl.when(j2 >= 1)
        def _():
          wait_out(p)

        transpose_block(p, 128)

        @pl.when(j < per_w - 2)
        def _():
          start_in(p, j + 2)

        start_out(p, j)
      return carry

    lax.fori_loop(0, per_w // 2, body, 0)
    wait_out(0)
    wait_out(1)

    if extra:

      @pl.when(w < extra)
      def _():
        start_in(0, per_w)
        wait_in(0)
        transpose_block(0, 128)
        start_out(0, per_w)
        wait_out(0)

    if rem:

      @pl.when(w == _NW - 1)
      def _():
        nt = rem * dim
        pltpu.sync_copy(tail_hbm, lin_v1.at[pl.ds(0, nt)])
        pltpu.sync_copy(
            lin_v1.at[pl.ds(0, nt)],
            out_hbm.at[pl.ds(nblk * 128 * dim, nt)],
        )

  return detile


@functools.lru_cache(maxsize=None)
def _make_gather_pack(vocab, seq, n_bt, dim):
  st_n = seq // 8
  mesh = plsc.VectorSubcoreMesh(core_axis_name="c", subcore_axis_name="s")

  @functools.partial(
      pl.kernel,
      mesh=mesh,
      out_type=jax.ShapeDtypeStruct((seq, dim // 8, n_bt, 1024),
                                    jnp.float32),
      compiler_params=pltpu.CompilerParams(
          use_tc_tiling_on_sc=False, needs_layout_passes=False
      ),
      scratch_types=[
          pltpu.VMEM((st_n, 8, 128), jnp.int32),
          pltpu.VMEM((2, 128, dim), jnp.float32),
          pltpu.VMEM((2, dim // 8, 1024), jnp.float32),
      ]
      + [pltpu.SemaphoreType.DMA] * 4,
  )
  def gather_pack(table_hbm, inp4_hbm, out_hbm, idx_v, rows_v, t_v, *sems):
    gsem = sems[:2]
    osem = sems[2:]
    w = lax.axis_index("s") * _NC + lax.axis_index("c")
    pltpu.sync_copy(inp4_hbm.at[pl.ds(0, st_n), w], idx_v)

    iota = _iota16()
    perm = [(iota + k) & 15 for k in range(16)]
    gk = [((iota + k) & 15) >> 3 for k in range(16)]
    jk = [(((iota + k) & 7) * 128) + iota for k in range(16)]

    def start_gather(p, s):
      pltpu.async_copy(
          table_hbm.at[idx_v.at[s // 8, s % 8]], rows_v.at[p], gsem[p]
      )

    def wait_gather(p):
      pltpu.make_async_copy(
          table_hbm.at[pl.ds(0, 128)], rows_v.at[p], gsem[p]
      ).wait()

    def start_write(p, s):
      pltpu.async_copy(
          t_v.at[p], out_hbm.at[s, pl.ds(0, dim // 8), w], osem[p]
      )

    def wait_write(p):
      pltpu.make_async_copy(
          out_hbm.at[0, pl.ds(0, dim // 8), 0], t_v.at[p], osem[p]
      ).wait()

    start_gather(0, 0)
    start_gather(1, 1)

    def body(s2, carry):
      for p in range(2):
        s = 2 * s2 + p
        wait_gather(p)

        @pl.when(s2 >= 1)
        def _():
          wait_write(p)

        def strip(ti, carry):
          t0 = ti * 16
          for d0 in range(0, dim, 16):
            for k in range(16):
              x = plsc.load_gather(
                  rows_v.at[p], [iota + t0, perm[k] + d0]
              )
              plsc.store_scatter(
                  t_v.at[p], [gk[k] + (d0 >> 3), jk[k] + t0], x
              )
          return carry

        lax.fori_loop(0, 8, strip, 0)

        @pl.when(s2 < seq // 2 - 1)
        def _():
          start_gather(p, s + 2)

        start_write(p, s)
      return carry

    lax.fori_loop(0, seq // 2, body, 0)
    wait_write(0)
    wait_write(1)

  return gather_pack


@functools.lru_cache(maxsize=None)
def _make_gather(vocab, dim, n, chunk, nbuf):
  n_per_w = n // _NW
  n_chunks = n_per_w // chunk
  n_groups = n_chunks // nbuf
  mesh = plsc.VectorSubcoreMesh(core_axis_name="c", subcore_axis_name="s")

  @functools.partial(
      pl.kernel,
      mesh=mesh,
      out_type=jax.ShapeDtypeStruct((n, dim), jnp.float32),
      compiler_params=pltpu.CompilerParams(use_tc_tiling_on_sc=False),
      scratch_types=[
          pltpu.VMEM((n_per_w,), jnp.int32),
          pltpu.VMEM((nbuf, chunk, dim), jnp.float32),
      ]
      + [pltpu.SemaphoreType.DMA] * (2 * nbuf),
  )
  def gather_kernel(table_hbm, idx_hbm, out_hbm, idx_v, rows_v, *sems):
    gsem = sems[:nbuf]
    osem = sems[nbuf:]
    wid = lax.axis_index("s") * _NC + lax.axis_index("c")
    base = wid * n_per_w
    pltpu.sync_copy(idx_hbm.at[pl.ds(base, n_per_w)], idx_v)

    def start_gather(b, c):
      pltpu.async_copy(
          table_hbm.at[idx_v.at[pl.ds(c * chunk, chunk)]],
          rows_v.at[b],
          gsem[b],
      )

    def wait_gather(b):
      pltpu.make_async_copy(
          table_hbm.at[pl.ds(0, chunk)], rows_v.at[b], gsem[b]
      ).wait()

    def start_out(b, c):
      pltpu.async_copy(
          rows_v.at[b], out_hbm.at[pl.ds(base + c * chunk, chunk)], osem[b]
      )

    def wait_out(b):
      pltpu.make_async_copy(
          out_hbm.at[pl.ds(base, chunk)], rows_v.at[b], osem[b]
      ).wait()

    for b in range(nbuf):
      start_gather(b, b)

    def group_body(g, carry):
      c0 = g * nbuf
      for b in range(nbuf):
        wait_gather(b)
        start_out(b, c0 + b)
      for b in range(nbuf):
        wait_out(b)
        start_gather(b, c0 + nbuf + b)
      return carry

    lax.fori_loop(0, n_groups - 1, group_body, 0)

    c0 = (n_groups - 1) * nbuf
    for b in range(nbuf):
      wait_gather(b)
      start_out(b, c0 + b)
    for b in range(nbuf):
      wait_out(b)

  return gather_kernel


def kernel(inp, table):
  b, s = inp.shape
  vocab, dim = table.shape
  n = b * s
  inp = inp.astype(jnp.int32)

  per_w = (vocab // 128) // _NW
  fast = (
      b == 128 * _NW
      and s % 8 == 0
      and dim == 32
      and vocab % 16 == 0
      and per_w >= 2
      and per_w % 2 == 0
  )
  if fast:
    n_bt = b // 128
    nblk = vocab // 128
    rem = vocab - nblk * 128
    if rem:
      tail = table[nblk * 128 :].reshape(rem * dim)
    else:
      tail = jnp.zeros((128,), jnp.float32)
    table_lin = _make_table_linear(vocab, dim)(table.T, tail)
    table_rm = table_lin.reshape(vocab, dim)
    inp4 = inp.T.reshape(s // 8, 8, n_bt, 128).transpose(0, 2, 1, 3)
    out4 = _make_gather_pack(vocab, s, n_bt, dim)(table_rm, inp4)
    out5 = out4.reshape(s, dim // 8, n_bt, 8, 128)
    return out5.transpose(2, 4, 0, 1, 3).reshape(b, s, dim)

  flat = inp.reshape(n)
  chunk, nbuf = 640, 5
  pad = (-n) % (_NW * chunk * nbuf)
  if pad:
    flat = jnp.concatenate([flat, jnp.zeros((pad,), jnp.int32)])
  rows = _make_gather(vocab, dim, n + pad, chunk, nbuf)(table, flat)
  if pad:
    rows = rows[:n]
  return rows.reshape(b, s, dim)
